# Initial kernel scaffold; baseline (speedup 1.0000x reference)
import jax
import jax.numpy as jnp
from jax.experimental import pallas as pl


def kernel(h, e, edge_index, enc_W, enc_b, post_W, post_b, bn_g, bn_b, r1_W, r1_b, r2_W, r2_b, r3_W, r3_b):
    raise NotImplementedError("write your pallas kernel here")


# speedup vs baseline: 8.6749x; 8.6749x over previous
import functools
import math

import jax
import jax.numpy as jnp
from jax import lax
from jax.experimental import pallas as pl
from jax.experimental.pallas import tpu as pltpu
from jax.experimental.pallas import tpu_sc as plsc

_AVG_D_LOG = math.log(33.0)

_NCORES = 2
_NSUB = 16
_NW = _NCORES * _NSUB
_CB = 256
_LANES = 16


def _sc_mesh():
    return plsc.VectorSubcoreMesh(
        core_axis_name="c", subcore_axis_name="s",
        num_cores=_NCORES, num_subcores=_NSUB)


def _wid():
    return lax.axis_index("s") * _NCORES + lax.axis_index("c")


def _lane_iota():
    return lax.broadcasted_iota(jnp.int32, (_LANES,), 0)


def _sread(ref, i):
    return ref[pl.ds(i, _LANES)][0]


def _swrite(ref, i, x, lane0):
    plsc.store_scatter(ref, [jnp.full((_LANES,), i, jnp.int32)],
                       jnp.full((_LANES,), x, ref.dtype), mask=lane0)


def _hist_kernel(E, NPAD):
    CH = E // _NW

    def body(dst_hbm, hist_hbm, dst_v, cnt_v, _sem):
        w = _wid()
        lane0 = _lane_iota() == 0
        wch = pl.multiple_of(w * CH, 8)
        pltpu.sync_copy(dst_hbm.at[pl.ds(wch, CH)], dst_v.at[pl.ds(0, CH)])

        def zero_body(i, _):
            cnt_v[pl.ds(i * _LANES, _LANES)] = jnp.zeros((_LANES,), jnp.int32)
            return 0
        lax.fori_loop(0, NPAD // _LANES, zero_body, 0)

        def chunk_body(q, _):
            dvec = dst_v[pl.ds(q * _LANES, _LANES)]
            for l in range(_LANES):
                d = dvec[l]
                c = _sread(cnt_v, d)
                _swrite(cnt_v, d, c + 1, lane0)
            return 0
        lax.fori_loop(0, CH // _LANES, chunk_body, 0)
        pltpu.sync_copy(cnt_v.at[pl.ds(0, NPAD)], hist_hbm.at[w])

    return pl.kernel(
        body,
        out_type=jax.ShapeDtypeStruct((_NW, NPAD), jnp.int32),
        mesh=_sc_mesh(),
        compiler_params=pltpu.CompilerParams(needs_layout_passes=False),
        scratch_types=[
            pltpu.VMEM((CH + _LANES,), jnp.int32),
            pltpu.VMEM((NPAD + _LANES,), jnp.int32),
            pltpu.SemaphoreType.DMA,
        ],
    )


def _offsets_kernel(N, E, NPAD):
    NR = NPAD // 128
    CH = E // _NW

    def body(hist_ref, off_ref, cin_ref, deg_ref, cur_ref, nb_ref):
        hist = hist_ref[...].astype(jnp.float32)
        tot = jnp.sum(hist, axis=0)
        li = lax.broadcasted_iota(jnp.int32, (128, 128), 0)
        lj = lax.broadcasted_iota(jnp.int32, (128, 128), 1)
        U = (li <= lj).astype(jnp.float32)
        ri = lax.broadcasted_iota(jnp.int32, (NR, NR), 0)
        rj = lax.broadcasted_iota(jnp.int32, (NR, NR), 1)
        Ls = (rj < ri).astype(jnp.float32)
        rowcum = jax.lax.dot(tot, U, precision=lax.Precision.HIGHEST)
        rowsum = jnp.sum(tot, axis=1, keepdims=True)
        rowpref = jax.lax.dot(
            Ls, jnp.broadcast_to(rowsum, (NR, 128)),
            precision=lax.Precision.HIGHEST)
        cin = rowcum + rowpref
        off = cin - tot
        off_ref[...] = off.astype(jnp.int32)
        cin_ref[...] = cin.astype(jnp.int32)
        deg_ref[...] = tot

        run = jnp.zeros((NR, 128), jnp.float32)
        for w in range(_NW):
            cur_ref[w] = (off + run).astype(jnp.int32)
            run = run + hist[w]

        fi = lax.broadcasted_iota(jnp.int32, (NR, 128), 0)
        fj = lax.broadcasted_iota(jnp.int32, (NR, 128), 1)
        valid = (fi * 128 + fj) < N
        rows = []
        for t in range(_NW + 1):
            cnt = jnp.sum(jnp.where(jnp.logical_and(cin <= float(t * CH), valid),
                                    1.0, 0.0))
            rows.append(jnp.full((1, 128), cnt, jnp.float32))
        nb_ref[...] = jnp.concatenate(rows, axis=0).astype(jnp.int32)

    return pl.pallas_call(
        body,
        out_shape=(
            jax.ShapeDtypeStruct((NR, 128), jnp.int32),
            jax.ShapeDtypeStruct((NR, 128), jnp.int32),
            jax.ShapeDtypeStruct((NR, 128), jnp.float32),
            jax.ShapeDtypeStruct((_NW, NR, 128), jnp.int32),
            jax.ShapeDtypeStruct((_NW + 1, 128), jnp.int32),
        ),
    )


def _place_kernel(N, E, NPAD):
    CH = E // _NW
    RB = 80
    NCH = CH // RB
    NG = RB // _LANES
    EP = E + _CB

    def body(src_hbm, dst_hbm, cur_hbm, ss_hbm, src_v, dst_v, cur_v,
             pos_v, zero_v, _sem):
        w = _wid()
        lane = _lane_iota()
        lane0 = lane == 0
        pltpu.sync_copy(src_hbm.at[pl.ds(w * CH, CH)], src_v)
        wch = pl.multiple_of(w * CH, 8)
        pltpu.sync_copy(dst_hbm.at[pl.ds(wch, CH)], dst_v.at[pl.ds(0, CH)])
        pltpu.sync_copy(cur_hbm.at[w], cur_v.at[pl.ds(0, NPAD)])

        @pl.when(w == 0)
        def _():
            def zb(i, _):
                zero_v[pl.ds(i * _LANES, _LANES)] = jnp.zeros((_LANES,),
                                                              jnp.int32)
                return 0
            lax.fori_loop(0, _CB // _LANES, zb, 0)
            pltpu.sync_copy(zero_v, ss_hbm.at[pl.ds(E, _CB)])

        def batch_body(j, _):
            for b in range(NG):
                dvec = dst_v[pl.ds(j * RB + b * _LANES, _LANES)]
                pos_vec = jnp.zeros((_LANES,), jnp.int32)
                for l in range(_LANES):
                    d = dvec[l]
                    p = _sread(cur_v, d)
                    _swrite(cur_v, d, p + 1, lane0)
                    pos_vec = jnp.where(lane == l, p, pos_vec)
                pos_v[j, pl.ds(b * _LANES, _LANES)] = pos_vec
            jrb = pl.multiple_of(j * RB, 8)
            pltpu.sync_copy(src_v.at[pl.ds(jrb, RB)],
                            ss_hbm.at[pos_v.at[j]])
            return 0
        lax.fori_loop(0, NCH, batch_body, 0)

    return pl.kernel(
        body,
        out_type=jax.ShapeDtypeStruct((EP,), jnp.int32),
        mesh=_sc_mesh(),
        compiler_params=pltpu.CompilerParams(needs_layout_passes=False),
        scratch_types=[
            pltpu.VMEM((CH,), jnp.int32),
            pltpu.VMEM((CH + _LANES,), jnp.int32),
            pltpu.VMEM((NPAD + _LANES,), jnp.int32),
            pltpu.VMEM((NCH, RB), jnp.int32),
            pltpu.VMEM((_CB,), jnp.int32),
            pltpu.SemaphoreType.DMA,
        ],
    )


def _agg_kernel(N, E, NPAD, EP):
    D = 128
    NVC = D // _LANES

    def body(x_hbm, ss_hbm, off_hbm, cin_hbm, nb_hbm, agg_hbm,
             off_v, cin_v, idx_v, rows_v, stage_v, nbv, sem):
        w = _wid()
        pltpu.sync_copy(off_hbm, off_v.at[pl.ds(0, NPAD)])
        pltpu.sync_copy(cin_hbm, cin_v.at[pl.ds(0, NPAD)])
        pltpu.sync_copy(nb_hbm.at[pl.ds(pl.multiple_of(w * 128, 8), _LANES)],
                        nbv.at[pl.ds(0, _LANES)])
        pltpu.sync_copy(nb_hbm.at[pl.ds(pl.multiple_of((w + 1) * 128, 8), _LANES)],
                        nbv.at[pl.ds(_LANES, _LANES)])
        nbvec = nbv[pl.ds(0, _LANES)]
        d0 = nbvec[0]
        nbvec2 = nbv[pl.ds(_LANES, _LANES)]
        d1 = nbvec2[0]

        zeros = jnp.zeros((_LANES,), jnp.float32)
        ninf = jnp.full((_LANES,), -jnp.inf, jnp.float32)
        pinf = jnp.full((_LANES,), jnp.inf, jnp.float32)

        def node_body(d, chunk_lo):
            s_e = _sread(off_v, d)
            e_end = _sread(cin_v, d)
            acc0 = ((zeros,) * NVC, (zeros,) * NVC,
                    (ninf,) * NVC, (pinf,) * NVC)

            def w_cond(st):
                return st[0] < e_end

            def w_body(st):
                e, clo, accs = st

                def refill():
                    e_al = pl.multiple_of(jnp.bitwise_and(e, -8), 8)
                    pltpu.sync_copy(ss_hbm.at[pl.ds(e_al, _CB)], idx_v)
                    pltpu.async_copy(x_hbm.at[idx_v], rows_v, sem).wait()
                    return e_al

                clo2 = lax.cond(e >= clo + _CB, refill, lambda: clo)
                k = jnp.minimum(e_end, clo2 + _CB) - e
                base = e - clo2

                def edge_body(j, a):
                    sums, sqs, mxs, mns = a
                    r = base + j
                    ns, nq, nx, nn = [], [], [], []
                    for c in range(NVC):
                        v = rows_v[r, pl.ds(c * _LANES, _LANES)]
                        ns.append(sums[c] + v)
                        nq.append(sqs[c] + v * v)
                        nx.append(jnp.maximum(mxs[c], v))
                        nn.append(jnp.minimum(mns[c], v))
                    return (tuple(ns), tuple(nq), tuple(nx), tuple(nn))

                accs2 = lax.fori_loop(0, k, edge_body, accs)
                return (e + k, clo2, accs2)

            _, chunk_lo2, accf = lax.while_loop(w_cond, w_body,
                                                (s_e, chunk_lo, acc0))
            sums, sqs, mxs, mns = accf
            for c in range(NVC):
                stage_v[pl.ds(c * _LANES, _LANES)] = sums[c]
                stage_v[pl.ds(D + c * _LANES, _LANES)] = sqs[c]
                stage_v[pl.ds(2 * D + c * _LANES, _LANES)] = mxs[c]
                stage_v[pl.ds(3 * D + c * _LANES, _LANES)] = mns[c]
            pltpu.sync_copy(stage_v, agg_hbm.at[d])
            return chunk_lo2

        lax.fori_loop(d0, d1, node_body, jnp.int32(-2 * _CB))

    return pl.kernel(
        body,
        out_type=jax.ShapeDtypeStruct((N, 4 * D), jnp.float32),
        mesh=_sc_mesh(),
        compiler_params=pltpu.CompilerParams(needs_layout_passes=False),
        scratch_types=[
            pltpu.VMEM((NPAD + _LANES,), jnp.int32),
            pltpu.VMEM((NPAD + _LANES,), jnp.int32),
            pltpu.VMEM((_CB,), jnp.int32),
            pltpu.VMEM((_CB, D), jnp.float32),
            pltpu.VMEM((4 * D,), jnp.float32),
            pltpu.VMEM((2 * _LANES,), jnp.int32),
            pltpu.SemaphoreType.DMA,
        ],
    )


def _matmul_kernel(N, BN):
    def body(x_ref, w_ref, b_ref, o_ref):
        o_ref[...] = jax.lax.dot(
            x_ref[...], w_ref[...],
            precision=lax.Precision.HIGHEST) + b_ref[...]

    D = 128
    return pl.pallas_call(
        body,
        grid=(N // BN,),
        in_specs=[
            pl.BlockSpec((BN, D), lambda i: (i, 0)),
            pl.BlockSpec((D, D), lambda i: (0, 0)),
            pl.BlockSpec((1, D), lambda i: (0, 0)),
        ],
        out_specs=pl.BlockSpec((BN, D), lambda i: (i, 0)),
        out_shape=jax.ShapeDtypeStruct((N, D), jnp.float32),
    )


def _dense_kernel(N, BN):
    D = 128
    NB = N // BN

    def body(x_ref, agg_ref, deg_ref, w0_ref, w1_ref, w2_ref, w3_ref, b_ref,
             pre_ref, st_ref, acc):
        i = pl.program_id(0)
        x = x_ref[...]
        agg = agg_ref[...]
        degr = deg_ref[...]
        ok = degr > 0.0
        degc = jnp.maximum(degr, 1.0)
        s = agg[:, 0:D]
        q = agg[:, D:2 * D]
        mx = agg[:, 2 * D:3 * D]
        mn = agg[:, 3 * D:4 * D]
        mean = jnp.where(ok, s / degc, 0.0)
        var = jnp.where(ok, q / degc - mean * mean, 0.0)
        std = jnp.sqrt(jax.nn.relu(var) + 1e-5)
        mx = jnp.where(ok, mx, 0.0)
        mn = jnp.where(ok, mn, 0.0)
        G = jnp.concatenate([mean, mx, mn, std], axis=1)
        delta = jnp.log(degc + 1.0) / _AVG_D_LOG
        delta = jnp.maximum(delta, 1e-5)
        hp = lax.Precision.HIGHEST
        pre = (jax.lax.dot(x, w0_ref[...], precision=hp)
               + jax.lax.dot(G, w1_ref[...], precision=hp)
               + jax.lax.dot(G * delta, w2_ref[...], precision=hp)
               + jax.lax.dot(G * (1.0 / delta), w3_ref[...], precision=hp)
               + b_ref[...])
        pre = jax.nn.relu(pre)
        pre_ref[...] = pre

        @pl.when(i == 0)
        def _():
            acc[...] = jnp.zeros_like(acc)
        acc[0:1] += jnp.sum(pre, axis=0, keepdims=True)
        acc[1:2] += jnp.sum(pre * pre, axis=0, keepdims=True)

        @pl.when(i == NB - 1)
        def _():
            st_ref[...] = acc[...]

    return pl.pallas_call(
        body,
        grid=(NB,),
        in_specs=[
            pl.BlockSpec((BN, D), lambda i: (i, 0)),
            pl.BlockSpec((BN, 4 * D), lambda i: (i, 0)),
            pl.BlockSpec((BN, 1), lambda i: (i, 0)),
            pl.BlockSpec((D, D), lambda i: (0, 0)),
            pl.BlockSpec((4 * D, D), lambda i: (0, 0)),
            pl.BlockSpec((4 * D, D), lambda i: (0, 0)),
            pl.BlockSpec((4 * D, D), lambda i: (0, 0)),
            pl.BlockSpec((1, D), lambda i: (0, 0)),
        ],
        out_specs=(
            pl.BlockSpec((BN, D), lambda i: (i, 0)),
            pl.BlockSpec((2, D), lambda i: (0, 0)),
        ),
        out_shape=(
            jax.ShapeDtypeStruct((N, D), jnp.float32),
            jax.ShapeDtypeStruct((2, D), jnp.float32),
        ),
        scratch_shapes=[pltpu.VMEM((2, D), jnp.float32)],
    )


def _norm_kernel(N, BN):
    D = 128

    def body(pre_ref, x_ref, st_ref, g_ref, beta_ref, o_ref):
        mu = st_ref[0:1] / float(N)
        var = st_ref[1:2] / float(N) - mu * mu
        inv = 1.0 / jnp.sqrt(var + 1e-5)
        o_ref[...] = ((pre_ref[...] - mu) * inv * g_ref[...] + beta_ref[...]
                      + x_ref[...])

    return pl.pallas_call(
        body,
        grid=(N // BN,),
        in_specs=[
            pl.BlockSpec((BN, D), lambda i: (i, 0)),
            pl.BlockSpec((BN, D), lambda i: (i, 0)),
            pl.BlockSpec((2, D), lambda i: (0, 0)),
            pl.BlockSpec((1, D), lambda i: (0, 0)),
            pl.BlockSpec((1, D), lambda i: (0, 0)),
        ],
        out_specs=pl.BlockSpec((BN, D), lambda i: (i, 0)),
        out_shape=jax.ShapeDtypeStruct((N, D), jnp.float32),
    )


def _readout_kernel(N, BN, NC):
    D = 128
    NB = N // BN

    def body(x_ref, w1_ref, b1_ref, w2_ref, b2_ref, w3_ref, b3_ref,
             o_ref, acc):
        i = pl.program_id(0)

        @pl.when(i == 0)
        def _():
            acc[...] = jnp.zeros_like(acc)
        acc[...] += jnp.sum(x_ref[...], axis=0, keepdims=True)

        @pl.when(i == NB - 1)
        def _():
            hp = lax.Precision.HIGHEST
            hg = acc[...] / float(N)
            z = jax.nn.relu(jax.lax.dot(hg, w1_ref[...], precision=hp)
                            + b1_ref[...])
            z = jax.nn.relu(jax.lax.dot(z, w2_ref[...], precision=hp)
                            + b2_ref[...])
            o_ref[...] = (jax.lax.dot(z, w3_ref[...], precision=hp)
                          + b3_ref[...])

    return pl.pallas_call(
        body,
        grid=(NB,),
        in_specs=[
            pl.BlockSpec((BN, D), lambda i: (i, 0)),
            pl.BlockSpec((D, D // 2), lambda i: (0, 0)),
            pl.BlockSpec((1, D // 2), lambda i: (0, 0)),
            pl.BlockSpec((D // 2, D // 4), lambda i: (0, 0)),
            pl.BlockSpec((1, D // 4), lambda i: (0, 0)),
            pl.BlockSpec((D // 4, NC), lambda i: (0, 0)),
            pl.BlockSpec((1, NC), lambda i: (0, 0)),
        ],
        out_specs=pl.BlockSpec((1, NC), lambda i: (0, 0)),
        out_shape=jax.ShapeDtypeStruct((1, NC), jnp.float32),
        scratch_shapes=[pltpu.VMEM((1, D), jnp.float32)],
    )


def kernel(h, e, edge_index, enc_W, enc_b, post_W, post_b, bn_g, bn_b,
           r1_W, r1_b, r2_W, r2_b, r3_W, r3_b):
    N, D = h.shape
    E = edge_index.shape[1]
    L = post_W.shape[0]
    NC = r3_b.shape[0]
    NPAD = ((N + 127) // 128) * 128
    NR = NPAD // 128
    EP = E + _CB
    BN = 1000
    assert E % _NW == 0 and (E // _NW) % 125 == 0 and N % BN == 0

    src = edge_index[0]
    dst = edge_index[1]

    hist = _hist_kernel(E, NPAD)(dst)
    off3, cin3, deg3, cur3, nb = _offsets_kernel(N, E, NPAD)(
        hist.reshape(_NW, NR, 128))
    off_f = off3.reshape(NPAD)
    cin_f = cin3.reshape(NPAD)
    deg = deg3.reshape(NPAD)[:N].reshape(N, 1)
    nb_f = nb.reshape((_NW + 1) * 128)
    ss = _place_kernel(N, E, NPAD)(src, dst, cur3.reshape(_NW, NPAD))

    x = _matmul_kernel(N, BN)(h, enc_W, enc_b.reshape(1, D))

    agg_fn = _agg_kernel(N, E, NPAD, EP)
    dense_fn = _dense_kernel(N, BN)
    norm_fn = _norm_kernel(N, BN)
    for l in range(L):
        agg = agg_fn(x, ss, off_f, cin_f, nb_f)
        W = post_W[l]
        pre, st = dense_fn(x, agg, deg, W[0:D], W[D:5 * D], W[5 * D:9 * D],
                           W[9 * D:13 * D], post_b[l].reshape(1, D))
        x = norm_fn(pre, x, st, bn_g[l].reshape(1, D), bn_b[l].reshape(1, D))

    return _readout_kernel(N, BN, NC)(
        x, r1_W, r1_b.reshape(1, -1), r2_W, r2_b.reshape(1, -1),
        r3_W, r3_b.reshape(1, -1))
